# baseline (speedup 1.0000x reference)
import jax
import jax.numpy as jnp
import numpy as np
from jax import lax
from jax.experimental import pallas as pl
from jax.experimental.pallas import tpu as pltpu
from jax.experimental.pallas import tpu_sc as plsc

B = 16384
D = 128
S = 25
C = 64
NC = 2
NS = 16
NW = NC * NS
PER_W = B // NW
K = 16
CHUNKS = PER_W // K
ROWS = K * S + K
GPC = 4
GLEN = ROWS // GPC
NLANE = 16
NVD = D // NLANE
DW = D // 2
HI = np.int32(-65536)
RB = np.int32(0x7FFF)


def _sc_body(idx_hbm, feat_hbm, self_out, sum_out,
             idx0, idx1, rows0, rows1, sum0, sum1, self0, self1,
             sem0, sem1, osem0, osem1):
    cid = lax.axis_index("c")
    sid = lax.axis_index("s")
    wid = sid * NC + cid
    idxs = (idx0, idx1)
    rows = (rows0, rows1)
    sums = (sum0, sum1)
    selfs = (self0, self1)
    sems = (sem0, sem1)
    osems = (osem0, osem1)

    def fire(c, b):
        t = wid * CHUNKS + c
        pltpu.sync_copy(idx_hbm.at[pl.ds(t * GPC, GPC)], idxs[b])
        for j in range(GPC):
            pltpu.async_copy(feat_hbm.at[idxs[b].at[j]],
                             rows[b].at[pl.ds(j * GLEN, GLEN)], sems[b])

    def drain(b):
        for j in range(GPC):
            pltpu.make_async_copy(feat_hbm.at[pl.ds(0, GLEN)],
                                  rows[b].at[pl.ds(j * GLEN, GLEN)],
                                  sems[b]).wait()

    def _rne(w):
        return w + RB + (lax.shift_right_logical(w, 16) & 1)

    def _pack_pair(e_f32, o_f32):
        e = _rne(lax.bitcast_convert_type(e_f32, jnp.int32))
        o = _rne(lax.bitcast_convert_type(o_f32, jnp.int32))
        return lax.shift_right_logical(e, 16) | (o & HI)

    def compute(c, b):
        rb = rows[b]
        sb = sums[b]
        fb = selfs[b]

        @plsc.parallel_loop(0, K, unroll=2)
        def _elem(k):
            r0 = k * S
            acc = tuple(rb[r0, pl.ds(NLANE * d, NLANE)] for d in range(NVD))

            def _sbody(s2, a):
                return tuple(a[d] + rb[r0 + s2, pl.ds(NLANE * d, NLANE)]
                             for d in range(NVD))

            acc = lax.fori_loop(1, S, _sbody, acc, unroll=8)
            for d in range(NVD // 2):
                sb[k, pl.ds(NLANE * d, NLANE)] = _pack_pair(acc[2 * d],
                                                            acc[2 * d + 1])
            rs = K * S + k
            for d in range(NVD // 2):
                fb[k, pl.ds(NLANE * d, NLANE)] = _pack_pair(
                    rb[rs, pl.ds(NLANE * 2 * d, NLANE)],
                    rb[rs, pl.ds(NLANE * (2 * d + 1), NLANE)])

        base = (wid * CHUNKS + c) * K
        pltpu.async_copy(fb, self_out.at[pl.ds(base, K)], osems[b])
        pltpu.async_copy(sb, sum_out.at[pl.ds(base, K)], osems[b])

    def drain_out(b):
        pltpu.make_async_copy(sum_out.at[pl.ds(0, K)], sums[b],
                              osems[b]).wait()
        pltpu.make_async_copy(self_out.at[pl.ds(0, K)], selfs[b],
                              osems[b]).wait()

    fire(0, 0)

    @pl.loop(0, CHUNKS, step=2)
    def _outer(cb):
        for b in range(2):
            c = cb + b

            @pl.when(c > 0)
            def _():
                drain_out(1 - b)

            @pl.when(c + 1 < CHUNKS)
            def _():
                fire(c + 1, 1 - b)

            drain(b)
            compute(c, b)

    drain_out(1)


def _sc_gather(idx_packed, features):
    f = pl.kernel(
        _sc_body,
        out_type=(jax.ShapeDtypeStruct((B, DW), jnp.int32),
                  jax.ShapeDtypeStruct((B, DW), jnp.int32)),
        mesh=plsc.VectorSubcoreMesh(core_axis_name="c", subcore_axis_name="s",
                                    num_cores=NC, num_subcores=NS),
        scratch_types=(
            pltpu.VMEM((GPC, GLEN), jnp.int32),
            pltpu.VMEM((GPC, GLEN), jnp.int32),
            pltpu.VMEM((ROWS, D), jnp.float32),
            pltpu.VMEM((ROWS, D), jnp.float32),
            pltpu.VMEM((K, DW), jnp.int32),
            pltpu.VMEM((K, DW), jnp.int32),
            pltpu.VMEM((K, DW), jnp.int32),
            pltpu.VMEM((K, DW), jnp.int32),
            pltpu.SemaphoreType.DMA,
            pltpu.SemaphoreType.DMA,
            pltpu.SemaphoreType.DMA,
            pltpu.SemaphoreType.DMA,
        ),
    )
    return f(idx_packed, features)


BM = 1024


_DN = (((1,), (1,)), ((), ()))

_PERM_E = np.concatenate([32 * d + np.arange(16) for d in range(4)]).astype(np.int32)
_PERM_O = np.concatenate([32 * d + 16 + np.arange(16) for d in range(4)]).astype(np.int32)


def _unpack2(w):
    pe = lax.bitcast_convert_type(w << 16, jnp.float32)
    po = lax.bitcast_convert_type(w & HI, jnp.float32)
    return pe, po


def _tc_body(xs_ref, xm_ref, a_ref, wgt_ref, o_ref):
    se, so = _unpack2(xs_ref[...])
    me, mo = _unpack2(xm_ref[...])
    h = jnp.dot(se, a_ref[0], preferred_element_type=jnp.float32)
    h = h + jnp.dot(so, a_ref[1], preferred_element_type=jnp.float32)
    h = h + jnp.dot(me, a_ref[2], preferred_element_type=jnp.float32)
    h = h + jnp.dot(mo, a_ref[3], preferred_element_type=jnp.float32)
    h = jnp.maximum(h, 0.0)
    o_ref[...] = lax.dot_general(h, wgt_ref[...], _DN,
                                 preferred_element_type=jnp.float32)


def _tc_head(xs, xm, a4, wgt):
    return pl.pallas_call(
        _tc_body,
        grid=(B // BM,),
        in_specs=[
            pl.BlockSpec((BM, DW), lambda i: (i, 0)),
            pl.BlockSpec((BM, DW), lambda i: (i, 0)),
            pl.BlockSpec((4, DW, D), lambda i: (0, 0, 0)),
            pl.BlockSpec((C, D), lambda i: (0, 0)),
        ],
        out_specs=pl.BlockSpec((BM, C), lambda i: (i, 0)),
        out_shape=jax.ShapeDtypeStruct((B, C), jnp.float32),
    )(xs, xm, a4, wgt)


def kernel(nodes, neigh_idx, features, W_enc, weight):
    idx_packed = jnp.concatenate(
        [neigh_idx.reshape(B // K, K * S), nodes.reshape(B // K, K)], axis=1
    ).reshape(-1, GLEN)
    self_pk, sum_pk = _sc_gather(idx_packed, features)
    wt_s = W_enc[:, :D].T
    wt_m = W_enc[:, D:].T * jnp.float32(1.0 / S)
    a4 = jnp.stack([wt_s[_PERM_E], wt_s[_PERM_O],
                    wt_m[_PERM_E], wt_m[_PERM_O]])
    return _tc_head(self_pk, sum_pk, a4, weight)


# speedup vs baseline: 1.0522x; 1.0522x over previous
import jax
import jax.numpy as jnp
from jax import lax
from jax.experimental import pallas as pl
from jax.experimental.pallas import tpu as pltpu
from jax.experimental.pallas import tpu_sc as plsc

B = 16384
D = 128
S = 25
C = 64
NC = 2
NS = 16
NW = NC * NS
PER_W = B // NW
K = 16
CHUNKS = PER_W // K
ROWS = K * S + K
GPC = 4
GLEN = (K * S) // GPC
NLANE = 16
NVD = D // NLANE


def _sc_body(nodes_hbm, neigh_hbm, feat_hbm, self_out, sum_out,
             idxs0, idxs1, idx0, idx1, rows0, rows1, sum0, sum1,
             sem0, sem1, osem0, osem1):
    cid = lax.axis_index("c")
    sid = lax.axis_index("s")
    wid = sid * NC + cid
    idxss = (idxs0, idxs1)
    idxs = (idx0, idx1)
    rows = (rows0, rows1)
    sums = (sum0, sum1)
    sems = (sem0, sem1)
    osems = (osem0, osem1)

    def fire(c, b):
        t = wid * CHUNKS + c
        pltpu.sync_copy(nodes_hbm.at[pl.ds(t * K, K)], idxss[b])
        pltpu.sync_copy(neigh_hbm.at[pl.ds(t * GPC, GPC)], idxs[b])
        for j in range(GPC):
            pltpu.async_copy(feat_hbm.at[idxs[b].at[j]],
                             rows[b].at[pl.ds(j * GLEN, GLEN)], sems[b])
        pltpu.async_copy(feat_hbm.at[idxss[b]], rows[b].at[pl.ds(K * S, K)],
                         sems[b])

    def drain(b):
        pltpu.make_async_copy(feat_hbm.at[pl.ds(0, ROWS)], rows[b],
                              sems[b]).wait()

    def compute(c, b):
        rb = rows[b]
        sb = sums[b]

        @plsc.parallel_loop(0, K, unroll=2)
        def _elem(k):
            r0 = k * S
            acc = tuple(rb[r0, pl.ds(NLANE * d, NLANE)] for d in range(NVD))

            def _sbody(s2, a):
                return tuple(a[d] + rb[r0 + s2, pl.ds(NLANE * d, NLANE)]
                             for d in range(NVD))

            acc = lax.fori_loop(1, S, _sbody, acc, unroll=8)
            for d in range(NVD):
                sb[k, pl.ds(NLANE * d, NLANE)] = acc[d]

        base = (wid * CHUNKS + c) * K
        pltpu.async_copy(rb.at[pl.ds(K * S, K)], self_out.at[pl.ds(base, K)],
                         osems[b])
        pltpu.async_copy(sb, sum_out.at[pl.ds(base, K)], osems[b])

    def drain_out(b):
        pltpu.make_async_copy(feat_hbm.at[pl.ds(0, K)], sums[b],
                              osems[b]).wait()
        pltpu.make_async_copy(feat_hbm.at[pl.ds(0, K)],
                              rows[b].at[pl.ds(K * S, K)], osems[b]).wait()

    fire(0, 0)

    @pl.loop(0, CHUNKS, step=2)
    def _outer(cb):
        for b in range(2):
            c = cb + b

            @pl.when(c > 0)
            def _():
                drain_out(1 - b)

            @pl.when(c + 1 < CHUNKS)
            def _():
                fire(c + 1, 1 - b)

            drain(b)
            compute(c, b)

    drain_out(1)


def _sc_gather(nodes, neigh2, features):
    f = pl.kernel(
        _sc_body,
        out_type=(jax.ShapeDtypeStruct((B, D), jnp.float32),
                  jax.ShapeDtypeStruct((B, D), jnp.float32)),
        mesh=plsc.VectorSubcoreMesh(core_axis_name="c", subcore_axis_name="s",
                                    num_cores=NC, num_subcores=NS),
        scratch_types=(
            pltpu.VMEM((K,), jnp.int32),
            pltpu.VMEM((K,), jnp.int32),
            pltpu.VMEM((GPC, GLEN), jnp.int32),
            pltpu.VMEM((GPC, GLEN), jnp.int32),
            pltpu.VMEM((ROWS, D), jnp.float32),
            pltpu.VMEM((ROWS, D), jnp.float32),
            pltpu.VMEM((K, D), jnp.float32),
            pltpu.VMEM((K, D), jnp.float32),
            pltpu.SemaphoreType.DMA,
            pltpu.SemaphoreType.DMA,
            pltpu.SemaphoreType.DMA,
            pltpu.SemaphoreType.DMA,
        ),
    )
    return f(nodes, neigh2, features)


BM = 4096


def _tc_body(xs_ref, xm_ref, ws_ref, wn_ref, wc_ref, o_ref):
    h = jnp.dot(xs_ref[...], ws_ref[...], preferred_element_type=jnp.float32)
    h = h + jnp.dot(xm_ref[...], wn_ref[...], preferred_element_type=jnp.float32)
    h = jnp.maximum(h, 0.0)
    o_ref[...] = jnp.dot(h, wc_ref[...], preferred_element_type=jnp.float32)


def _tc_head(xs, xm, ws_t, wn_t, wc_t):
    return pl.pallas_call(
        _tc_body,
        grid=(B // BM,),
        in_specs=[
            pl.BlockSpec((BM, D), lambda i: (i, 0)),
            pl.BlockSpec((BM, D), lambda i: (i, 0)),
            pl.BlockSpec((D, D), lambda i: (0, 0)),
            pl.BlockSpec((D, D), lambda i: (0, 0)),
            pl.BlockSpec((D, C), lambda i: (0, 0)),
        ],
        out_specs=pl.BlockSpec((BM, C), lambda i: (i, 0)),
        out_shape=jax.ShapeDtypeStruct((B, C), jnp.float32),
    )(xs, xm, ws_t, wn_t, wc_t)


def kernel(nodes, neigh_idx, features, W_enc, weight):
    self_out, sum_out = _sc_gather(nodes, neigh_idx.reshape(-1, GLEN), features)
    ws_t = W_enc[:, :D].T
    wn_t = W_enc[:, D:].T * jnp.float32(1.0 / S)
    wc_t = weight.T
    return _tc_head(self_out, sum_out, ws_t, wn_t, wc_t)


# speedup vs baseline: 1.0956x; 1.0412x over previous
import jax
import jax.numpy as jnp
from jax import lax
from jax.experimental import pallas as pl
from jax.experimental.pallas import tpu as pltpu
from jax.experimental.pallas import tpu_sc as plsc

B = 16384
D = 128
S = 25
C = 64
NC = 2
NS = 16
NW = NC * NS
PER_W = B // NW
K = 16
CHUNKS = PER_W // K
ROWS = K * S + K
GPC = 4
GLEN = ROWS // GPC
NLANE = 16
NVD = D // NLANE


def _sc_body(idx_hbm, feat_hbm, self_out, sum_out,
             idx0, idx1, rows0, rows1, sum0, sum1,
             sem0, sem1, osem0, osem1):
    cid = lax.axis_index("c")
    sid = lax.axis_index("s")
    wid = sid * NC + cid
    idxs = (idx0, idx1)
    rows = (rows0, rows1)
    sums = (sum0, sum1)
    sems = (sem0, sem1)
    osems = (osem0, osem1)

    def fire(c, b):
        t = wid * CHUNKS + c
        pltpu.sync_copy(idx_hbm.at[pl.ds(t * GPC, GPC)], idxs[b])
        for j in range(GPC):
            pltpu.async_copy(feat_hbm.at[idxs[b].at[j]],
                             rows[b].at[pl.ds(j * GLEN, GLEN)], sems[b])

    def drain(b):
        for j in range(GPC):
            pltpu.make_async_copy(feat_hbm.at[pl.ds(0, GLEN)],
                                  rows[b].at[pl.ds(j * GLEN, GLEN)],
                                  sems[b]).wait()

    def compute(c, b):
        rb = rows[b]
        sb = sums[b]

        @plsc.parallel_loop(0, K, unroll=2)
        def _elem(k):
            r0 = k * S
            acc = tuple(rb[r0, pl.ds(NLANE * d, NLANE)] for d in range(NVD))

            def _sbody(s2, a):
                return tuple(a[d] + rb[r0 + s2, pl.ds(NLANE * d, NLANE)]
                             for d in range(NVD))

            acc = lax.fori_loop(1, S, _sbody, acc, unroll=8)
            for d in range(NVD):
                sb[k, pl.ds(NLANE * d, NLANE)] = acc[d]

        base = (wid * CHUNKS + c) * K
        pltpu.async_copy(rb.at[pl.ds(K * S, K)], self_out.at[pl.ds(base, K)],
                         osems[b])
        pltpu.async_copy(sb, sum_out.at[pl.ds(base, K)], osems[b])

    def drain_out(b):
        pltpu.make_async_copy(feat_hbm.at[pl.ds(0, K)], sums[b],
                              osems[b]).wait()
        pltpu.make_async_copy(feat_hbm.at[pl.ds(0, K)],
                              rows[b].at[pl.ds(K * S, K)], osems[b]).wait()

    fire(0, 0)

    @pl.loop(0, CHUNKS, step=2)
    def _outer(cb):
        for b in range(2):
            c = cb + b

            @pl.when(c > 0)
            def _():
                drain_out(1 - b)

            @pl.when(c + 1 < CHUNKS)
            def _():
                fire(c + 1, 1 - b)

            drain(b)
            compute(c, b)

    drain_out(1)


def _sc_gather(idx_packed, features):
    f = pl.kernel(
        _sc_body,
        out_type=(jax.ShapeDtypeStruct((B, D), jnp.float32),
                  jax.ShapeDtypeStruct((B, D), jnp.float32)),
        mesh=plsc.VectorSubcoreMesh(core_axis_name="c", subcore_axis_name="s",
                                    num_cores=NC, num_subcores=NS),
        scratch_types=(
            pltpu.VMEM((GPC, GLEN), jnp.int32),
            pltpu.VMEM((GPC, GLEN), jnp.int32),
            pltpu.VMEM((ROWS, D), jnp.float32),
            pltpu.VMEM((ROWS, D), jnp.float32),
            pltpu.VMEM((K, D), jnp.float32),
            pltpu.VMEM((K, D), jnp.float32),
            pltpu.SemaphoreType.DMA,
            pltpu.SemaphoreType.DMA,
            pltpu.SemaphoreType.DMA,
            pltpu.SemaphoreType.DMA,
        ),
    )
    return f(idx_packed, features)


BM = 8192


def _tc_body(xs_ref, xm_ref, ws_ref, wn_ref, wc_ref, o_ref):
    h = jnp.dot(xs_ref[...], ws_ref[...], preferred_element_type=jnp.float32)
    h = h + jnp.dot(xm_ref[...], wn_ref[...], preferred_element_type=jnp.float32)
    h = jnp.maximum(h, 0.0)
    o_ref[...] = jnp.dot(h, wc_ref[...], preferred_element_type=jnp.float32)


def _tc_head(xs, xm, ws_t, wn_t, wc_t):
    return pl.pallas_call(
        _tc_body,
        grid=(B // BM,),
        in_specs=[
            pl.BlockSpec((BM, D), lambda i: (i, 0)),
            pl.BlockSpec((BM, D), lambda i: (i, 0)),
            pl.BlockSpec((D, D), lambda i: (0, 0)),
            pl.BlockSpec((D, D), lambda i: (0, 0)),
            pl.BlockSpec((D, C), lambda i: (0, 0)),
        ],
        out_specs=pl.BlockSpec((BM, C), lambda i: (i, 0)),
        out_shape=jax.ShapeDtypeStruct((B, C), jnp.float32),
    )(xs, xm, ws_t, wn_t, wc_t)


def kernel(nodes, neigh_idx, features, W_enc, weight):
    idx_packed = jnp.concatenate(
        [neigh_idx.reshape(B // K, K * S), nodes.reshape(B // K, K)], axis=1
    ).reshape(-1, GLEN)
    self_out, sum_out = _sc_gather(idx_packed, features)
    ws_t = W_enc[:, :D].T
    wn_t = W_enc[:, D:].T * jnp.float32(1.0 / S)
    wc_t = weight.T
    return _tc_head(self_out, sum_out, ws_t, wn_t, wc_t)
